# baseline (speedup 1.0000x reference)
import functools

import jax
import jax.numpy as jnp
from jax import lax
from jax.experimental import pallas as pl
from jax.experimental.pallas import tpu as pltpu
from jax.experimental.pallas import tpu_sc as plsc

N = 10000
E = 320000
D = 128
H = 16
C = 40

NC = 2
NS = 16
L = 16
NW = NC * NS
CH = 128
NBLK = E // CH
BPW = NBLK // NW
NEXTRA = NBLK - BPW * NW
NP = 10240
G = NP // 8
GN = N // 8
RPT = NP // NS
ZROWS = 128


def _rand_noise(shape, seed):
    pltpu.prng_seed(seed)
    bits = pltpu.prng_random_bits(shape).astype(jnp.uint32)
    return (bits >> 8).astype(jnp.float32) * (2.0 ** -24) - 0.5


def _qd_from_stats(x, rmin, rmax, noise):
    denom = jnp.maximum(rmax - rmin, 1e-6)
    rscale = 255.0 / denom
    q = (x - rmin) * rscale
    q = jnp.clip(jnp.round(q + noise), 0.0, 255.0)
    return q / rscale + rmin


def _seg_stats(x, w):
    mins = []
    maxs = []
    for m in range(8):
        seg = x[:, w * m:w * (m + 1)]
        mn = jnp.min(seg, axis=1, keepdims=True)
        mx = jnp.max(seg, axis=1, keepdims=True)
        mins.append(jnp.broadcast_to(mn, seg.shape))
        maxs.append(jnp.broadcast_to(mx, seg.shape))
    return jnp.concatenate(mins, axis=1), jnp.concatenate(maxs, axis=1)


def _qd_mm_body(x_ref, w8_ref, o_ref):
    x = x_ref[...]
    noise = _rand_noise(x.shape, 12301)
    rmin, rmax = _seg_stats(x, 128)
    dq = _qd_from_stats(x, rmin, rmax, noise)
    o_ref[...] = jnp.dot(dq, w8_ref[...], preferred_element_type=jnp.float32)


def _combine_relu_qd_mm_body(zp_ref, w8_ref, o_ref):
    h = jnp.maximum(zp_ref[0] + zp_ref[1], 0.0)
    noise = _rand_noise(h.shape, 12302)
    rmin, rmax = _seg_stats(h, 16)
    dq = _qd_from_stats(h, rmin, rmax, noise)
    o_ref[...] = jnp.dot(dq, w8_ref[...], preferred_element_type=jnp.float32)


def _combine_relu_qd_body(zp_ref, o_ref):
    h = jnp.maximum(zp_ref[0] + zp_ref[1], 0.0)
    noise = _rand_noise(h.shape, 12303)
    rmin, rmax = _seg_stats(h, 16)
    o_ref[...] = _qd_from_stats(h, rmin, rmax, noise)


def _combine_mm_body(zp_ref, w8_ref, o_ref):
    z = zp_ref[0] + zp_ref[1]
    o_ref[...] = jnp.dot(z, w8_ref[...], preferred_element_type=jnp.float32)


def _qd_mm(xg, w8):
    return pl.pallas_call(
        _qd_mm_body,
        out_shape=jax.ShapeDtypeStruct((GN, 128), jnp.float32),
    )(xg, w8)


def _combine_relu_qd_mm(zp, w8):
    return pl.pallas_call(
        _combine_relu_qd_mm_body,
        out_shape=jax.ShapeDtypeStruct((G, 128), jnp.float32),
    )(zp, w8)


def _combine_relu_qd(zp):
    return pl.pallas_call(
        _combine_relu_qd_body,
        out_shape=jax.ShapeDtypeStruct((G, 128), jnp.float32),
    )(zp)


def _combine_mm(zp, w8):
    return pl.pallas_call(
        _combine_mm_body,
        out_shape=jax.ShapeDtypeStruct((G, 8 * 48), jnp.float32),
    )(zp, w8)


def _spmm_sc_kernel(y_hbm, ev_hbm, out_hbm,
                    idx, rows, zbuf, zsh, sem_i, sem_g, sem_a):
    c = lax.axis_index("c")
    s = lax.axis_index("s")
    wid = s * NC + c
    bstart = wid * BPW
    nextra = jnp.where(wid < NEXTRA, 1, 0)

    ci = pltpu.async_copy(ev_hbm.at[pl.ds(bstart, BPW)], idx.at[pl.ds(0, BPW)],
                          sem_i)
    cx = pltpu.async_copy(
        ev_hbm.at[pl.ds(NW * BPW + jnp.minimum(wid, NEXTRA - 1), 1)],
        idx.at[pl.ds(BPW, 1)], sem_i)

    def zfill(i, _):
        zbuf[i, :] = jnp.zeros((L,), jnp.float32)
        return 0
    lax.fori_loop(0, ZROWS, zfill, 0)

    def zcopy(j, _):
        pltpu.sync_copy(zbuf, zsh.at[pl.ds(s * RPT + j * ZROWS, ZROWS)])
        return 0
    lax.fori_loop(0, RPT // ZROWS, zcopy, 0)
    ci.wait()
    cx.wait()
    plsc.subcore_barrier()

    @pl.when(nextra > 0)
    def _():
        pltpu.async_copy(y_hbm.at[idx.at[BPW, 0]], rows.at[0], sem_g).wait()
        pltpu.sync_copy(rows.at[0], zsh.at[idx.at[BPW, 1]], add=True)

    pltpu.async_copy(y_hbm.at[idx.at[0, 0]], rows.at[0], sem_g)
    pltpu.async_copy(y_hbm.at[idx.at[1, 0]], rows.at[1], sem_g)

    def chunk(g, _):
        b = lax.rem(g, 3)
        pltpu.make_async_copy(y_hbm.at[idx.at[g, 0]], rows.at[b], sem_g).wait()

        @pl.when(g + 2 < BPW)
        def _():
            bn = lax.rem(g + 2, 3)

            @pl.when(g >= 1)
            def _():
                pltpu.make_async_copy(y_hbm.at[idx.at[g, 0]], rows.at[bn],
                                      sem_a).wait()
            pltpu.async_copy(y_hbm.at[idx.at[g + 2, 0]], rows.at[bn], sem_g)

        pltpu.async_copy(rows.at[b], zsh.at[idx.at[g, 1]], sem_a, add=True)
        return 0
    lax.fori_loop(0, BPW, chunk, 0)
    pltpu.make_async_copy(y_hbm.at[idx.at[0, 0]], rows.at[0], sem_a).wait()
    pltpu.make_async_copy(y_hbm.at[idx.at[0, 0]], rows.at[0], sem_a).wait()
    pltpu.make_async_copy(y_hbm.at[idx.at[0, 0]], rows.at[0], sem_a).wait()
    plsc.subcore_barrier()

    pltpu.sync_copy(zsh.at[pl.ds(s * RPT, RPT)],
                    out_hbm.at[c].at[pl.ds(s * RPT, RPT)])


def _spmm_sc(y, ev):
    mesh = plsc.VectorSubcoreMesh(core_axis_name="c", subcore_axis_name="s")
    k = functools.partial(
        pl.kernel,
        out_type=jax.ShapeDtypeStruct((NC, NP, L), jnp.float32),
        mesh=mesh,
        scratch_types=[
            pltpu.VMEM((BPW + 1, 2, CH), jnp.int32),
            pltpu.VMEM((3, CH, L), jnp.float32),
            pltpu.VMEM((ZROWS, L), jnp.float32),
            pltpu.VMEM_SHARED((NP, L), jnp.float32),
            pltpu.SemaphoreType.DMA,
            pltpu.SemaphoreType.DMA,
            pltpu.SemaphoreType.DMA,
        ],
        compiler_params=pltpu.CompilerParams(use_tc_tiling_on_sc=False),
    )(_spmm_sc_kernel)
    return k(y, ev)


def kernel(features, edge_index, W1, W2, W3):
    ev = edge_index.reshape(2, NBLK, CH).transpose(1, 0, 2)
    eye8 = jnp.eye(8, dtype=jnp.float32)
    W1g = jnp.kron(eye8, W1)
    W2g = jnp.kron(eye8, W2)
    W3p = jnp.pad(W3, ((0, 0), (0, 8)))
    W3g = jnp.kron(eye8, W3p)

    y1 = _qd_mm(features.reshape(GN, 8 * D), W1g)
    z1 = _spmm_sc(y1.reshape(N, H), ev)
    y2 = _combine_relu_qd_mm(z1.reshape(NC, G, 128), W2g)
    z2 = _spmm_sc(y2.reshape(NP, H), ev)
    y3 = _combine_relu_qd(z2.reshape(NC, G, 128))
    z3 = _spmm_sc(y3.reshape(NP, H), ev)
    outv = _combine_mm(z3.reshape(NC, G, 128), W3g)
    return outv.reshape(NP, 48)[:N, :C]


# speedup vs baseline: 1.0087x; 1.0087x over previous
import functools

import jax
import jax.numpy as jnp
from jax import lax
from jax.experimental import pallas as pl
from jax.experimental.pallas import tpu as pltpu
from jax.experimental.pallas import tpu_sc as plsc

N = 10000
E = 320000
D = 128
H = 16
C = 40

NC = 2
NS = 16
L = 16
NW = NC * NS
CH = 128
NBLK = E // CH
BPW = NBLK // NW
NEXTRA = NBLK - BPW * NW
NP = 10240
G = NP // 8
GN = N // 8
RPT = NP // NS
ZROWS = 128


def _rand_noise(shape, seed):
    pltpu.prng_seed(seed)
    bits = pltpu.prng_random_bits(shape).astype(jnp.uint32)
    return (bits >> 8).astype(jnp.float32) * (2.0 ** -24) - 0.5


def _qd_from_stats(x, rmin, rmax, noise):
    denom = jnp.maximum(rmax - rmin, 1e-6)
    rscale = 255.0 / denom
    q = (x - rmin) * rscale
    q = jnp.clip(jnp.round(q + noise), 0.0, 255.0)
    return q / rscale + rmin


def _seg_stats(x, w):
    mins = []
    maxs = []
    for m in range(8):
        seg = x[:, w * m:w * (m + 1)]
        mn = jnp.min(seg, axis=1, keepdims=True)
        mx = jnp.max(seg, axis=1, keepdims=True)
        mins.append(jnp.broadcast_to(mn, seg.shape))
        maxs.append(jnp.broadcast_to(mx, seg.shape))
    return jnp.concatenate(mins, axis=1), jnp.concatenate(maxs, axis=1)


def _qd_mm_body(x_ref, w_ref, o_ref):
    x = x_ref[...]
    noise = _rand_noise(x.shape, 12301)
    rmin = jnp.min(x, axis=1, keepdims=True)
    rmax = jnp.max(x, axis=1, keepdims=True)
    dq = _qd_from_stats(x, rmin, rmax, noise)
    o_ref[...] = jnp.dot(dq, w_ref[...], preferred_element_type=jnp.float32)


def _combine_relu_qd_mm_body(zp_ref, w8_ref, o_ref):
    h = jnp.maximum(zp_ref[0] + zp_ref[1], 0.0)
    noise = _rand_noise(h.shape, 12302)
    rmin, rmax = _seg_stats(h, 16)
    dq = _qd_from_stats(h, rmin, rmax, noise)
    o_ref[...] = jnp.dot(dq, w8_ref[...], preferred_element_type=jnp.float32)


def _combine_relu_qd_body(zp_ref, o_ref):
    h = jnp.maximum(zp_ref[0] + zp_ref[1], 0.0)
    noise = _rand_noise(h.shape, 12303)
    rmin, rmax = _seg_stats(h, 16)
    o_ref[...] = _qd_from_stats(h, rmin, rmax, noise)


def _combine_mm_body(zp_ref, w_ref, o_ref):
    z = zp_ref[0] + zp_ref[1]
    ot = jax.lax.dot_general(w_ref[...], z, (((0,), (1,)), ((), ())),
                             preferred_element_type=jnp.float32)
    o_ref[...] = ot[:, :N]


def _qd_mm(x, w):
    return pl.pallas_call(
        _qd_mm_body,
        out_shape=jax.ShapeDtypeStruct((N, H), jnp.float32),
    )(x, w)


def _combine_relu_qd_mm(zp, w8):
    return pl.pallas_call(
        _combine_relu_qd_mm_body,
        out_shape=jax.ShapeDtypeStruct((G, 128), jnp.float32),
    )(zp, w8)


def _combine_relu_qd(zp):
    return pl.pallas_call(
        _combine_relu_qd_body,
        out_shape=jax.ShapeDtypeStruct((G, 128), jnp.float32),
    )(zp)


def _combine_mm(zp, w):
    return pl.pallas_call(
        _combine_mm_body,
        out_shape=jax.ShapeDtypeStruct((C, N), jnp.float32),
    )(zp, w)


def _spmm_sc_kernel(y_hbm, ev_hbm, out_hbm,
                    idx, rows, zbuf, zsh, sem_i, sem_g, sem_a):
    c = lax.axis_index("c")
    s = lax.axis_index("s")
    wid = s * NC + c
    bstart = wid * BPW
    nextra = jnp.where(wid < NEXTRA, 1, 0)

    ci = pltpu.async_copy(ev_hbm.at[pl.ds(bstart, BPW)], idx.at[pl.ds(0, BPW)],
                          sem_i)
    cx = pltpu.async_copy(
        ev_hbm.at[pl.ds(NW * BPW + jnp.minimum(wid, NEXTRA - 1), 1)],
        idx.at[pl.ds(BPW, 1)], sem_i)

    def zfill(i, _):
        zbuf[i, :] = jnp.zeros((L,), jnp.float32)
        return 0
    lax.fori_loop(0, ZROWS, zfill, 0)

    def zcopy(j, _):
        pltpu.sync_copy(zbuf, zsh.at[pl.ds(s * RPT + j * ZROWS, ZROWS)])
        return 0
    lax.fori_loop(0, RPT // ZROWS, zcopy, 0)
    ci.wait()
    cx.wait()
    plsc.subcore_barrier()

    @pl.when(nextra > 0)
    def _():
        pltpu.async_copy(y_hbm.at[idx.at[BPW, 0]], rows.at[0], sem_g).wait()
        pltpu.sync_copy(rows.at[0], zsh.at[idx.at[BPW, 1]], add=True)

    pltpu.async_copy(y_hbm.at[idx.at[0, 0]], rows.at[0], sem_g)
    pltpu.async_copy(y_hbm.at[idx.at[1, 0]], rows.at[1], sem_g)

    def chunk(g, _):
        b = lax.rem(g, 3)
        pltpu.make_async_copy(y_hbm.at[idx.at[g, 0]], rows.at[b], sem_g).wait()

        @pl.when(g + 2 < BPW)
        def _():
            bn = lax.rem(g + 2, 3)

            @pl.when(g >= 1)
            def _():
                pltpu.make_async_copy(y_hbm.at[idx.at[g, 0]], rows.at[bn],
                                      sem_a).wait()
            pltpu.async_copy(y_hbm.at[idx.at[g + 2, 0]], rows.at[bn], sem_g)

        pltpu.async_copy(rows.at[b], zsh.at[idx.at[g, 1]], sem_a, add=True)
        return 0
    lax.fori_loop(0, BPW, chunk, 0)
    pltpu.make_async_copy(y_hbm.at[idx.at[0, 0]], rows.at[0], sem_a).wait()
    pltpu.make_async_copy(y_hbm.at[idx.at[0, 0]], rows.at[0], sem_a).wait()
    pltpu.make_async_copy(y_hbm.at[idx.at[0, 0]], rows.at[0], sem_a).wait()
    plsc.subcore_barrier()

    pltpu.sync_copy(zsh.at[pl.ds(s * RPT, RPT)],
                    out_hbm.at[c].at[pl.ds(s * RPT, RPT)])


def _spmm_sc(y, ev):
    mesh = plsc.VectorSubcoreMesh(core_axis_name="c", subcore_axis_name="s")
    k = functools.partial(
        pl.kernel,
        out_type=jax.ShapeDtypeStruct((NC, NP, L), jnp.float32),
        mesh=mesh,
        scratch_types=[
            pltpu.VMEM((BPW + 1, 2, CH), jnp.int32),
            pltpu.VMEM((3, CH, L), jnp.float32),
            pltpu.VMEM((ZROWS, L), jnp.float32),
            pltpu.VMEM_SHARED((NP, L), jnp.float32),
            pltpu.SemaphoreType.DMA,
            pltpu.SemaphoreType.DMA,
            pltpu.SemaphoreType.DMA,
        ],
        compiler_params=pltpu.CompilerParams(use_tc_tiling_on_sc=False),
    )(_spmm_sc_kernel)
    return k(y, ev)


def kernel(features, edge_index, W1, W2, W3):
    ev = edge_index.reshape(2, NBLK, CH).transpose(1, 0, 2)
    eye8 = jnp.eye(8, dtype=jnp.float32)
    W2g = jnp.kron(eye8, W2)

    y1 = _qd_mm(features, W1)
    z1 = _spmm_sc(y1, ev)
    y2 = _combine_relu_qd_mm(z1.reshape(NC, G, 128), W2g)
    z2 = _spmm_sc(y2.reshape(NP, H), ev)
    y3 = _combine_relu_qd(z2.reshape(NC, G, 128))
    z3 = _spmm_sc(y3.reshape(NP, H), ev)
    outt = _combine_mm(z3, W3)
    return outt.T
